# Initial kernel scaffold; baseline (speedup 1.0000x reference)
import jax
import jax.numpy as jnp
from jax.experimental import pallas as pl


def kernel(x):
    raise NotImplementedError("write your pallas kernel here")


# speedup vs baseline: 20.9096x; 20.9096x over previous
import jax
import jax.numpy as jnp
from jax.experimental import pallas as pl
from jax.experimental.pallas import tpu as pltpu

_K = 8
_NEG = float("-inf")


def _top8_desc(w):
    s, f = w.shape
    rows = jax.lax.broadcasted_iota(jnp.int32, (s, f), 0)
    outs = []
    for _ in range(_K):
        m = jnp.max(w, axis=0, keepdims=True)
        outs.append(m)
        is_max = w == m
        idx = jnp.where(is_max, rows, s)
        amin = jnp.min(idx, axis=0, keepdims=True)
        w = jnp.where(rows == amin, _NEG, w)
    return jnp.concatenate(outs, axis=0)


def _body(x_ref, o_ref, acc_ref):
    c = pl.program_id(1)
    t8 = _top8_desc(x_ref[0])

    @pl.when(c == 0)
    def _():
        acc_ref[...] = t8

    @pl.when(c > 0)
    def _():
        acc_ref[...] = _top8_desc(
            jnp.concatenate([acc_ref[...], t8], axis=0))

    @pl.when(c == pl.num_programs(1) - 1)
    def _():
        o_ref[0] = acc_ref[...]


def kernel(x):
    b, s, f = x.shape
    chunk = min(4096, s)
    assert s % chunk == 0
    grid = (b, s // chunk)
    return pl.pallas_call(
        _body,
        grid=grid,
        in_specs=[pl.BlockSpec((1, chunk, f), lambda i, c: (i, c, 0))],
        out_specs=pl.BlockSpec((1, _K, f), lambda i, c: (i, 0, 0)),
        out_shape=jax.ShapeDtypeStruct((b, _K, f), x.dtype),
        scratch_shapes=[pltpu.VMEM((_K, f), x.dtype)],
        compiler_params=pltpu.CompilerParams(
            dimension_semantics=("arbitrary", "arbitrary")),
    )(x)


# speedup vs baseline: 26.0366x; 1.2452x over previous
import jax
import jax.numpy as jnp
from jax.experimental import pallas as pl
from jax.experimental.pallas import tpu as pltpu

_K = 8
_NEG = float("-inf")


def _top8_desc(w):
    s, f = w.shape
    rows = jax.lax.broadcasted_iota(jnp.int32, (s, f), 0)
    outs = []
    for _ in range(_K):
        m = jnp.max(w, axis=0, keepdims=True)
        outs.append(m)
        is_max = w == m
        idx = jnp.where(is_max, rows, s)
        amin = jnp.min(idx, axis=0, keepdims=True)
        w = jnp.where(rows == amin, _NEG, w)
    return jnp.concatenate(outs, axis=0)


def _body(f, x_ref, o_ref, acc_ref):
    c = pl.program_id(1)
    t8 = _top8_desc(x_ref[0])

    @pl.when(c == 0)
    def _():
        acc_ref[...] = t8

    @pl.when(c > 0)
    def _():
        acc_ref[...] = _top8_desc(
            jnp.concatenate([acc_ref[...], t8], axis=0))

    @pl.when(c == pl.num_programs(1) - 1)
    def _():
        acc = acc_ref[...]
        o_ref[0] = _top8_desc(
            jnp.concatenate([acc[:, :f], acc[:, f:]], axis=0))


def kernel(x):
    b, s, f = x.shape
    xr = x.reshape(b, s // 2, 2 * f)
    chunk = min(4096, s // 2)
    assert (s // 2) % chunk == 0
    grid = (b, (s // 2) // chunk)
    import functools
    return pl.pallas_call(
        functools.partial(_body, f),
        grid=grid,
        in_specs=[pl.BlockSpec((1, chunk, 2 * f), lambda i, c: (i, c, 0))],
        out_specs=pl.BlockSpec((1, _K, f), lambda i, c: (i, 0, 0)),
        out_shape=jax.ShapeDtypeStruct((b, _K, f), x.dtype),
        scratch_shapes=[pltpu.VMEM((_K, 2 * f), x.dtype)],
        compiler_params=pltpu.CompilerParams(
            dimension_semantics=("arbitrary", "arbitrary")),
    )(xr)


# speedup vs baseline: 35.6770x; 1.3703x over previous
import dataclasses
import functools

import jax
import jax.numpy as jnp
from jax import lax
from jax.experimental import pallas as pl
from jax.experimental.pallas import tpu as pltpu
from jax.experimental.pallas import tpu_sc as plsc

_K = 8
_NEG = float("-inf")

_B, _S, _F = 32, 32768, 64
_G = 2048
_GSZ = _S // _G
_CH = 8192
_NPAIR = _B * _F
_CAND = _K * _GSZ


def _sc_params():
    cp = pltpu.CompilerParams(use_tc_tiling_on_sc=False)
    if "needs_layout_passes" in pltpu.CompilerParams.__dataclass_fields__:
        cp = dataclasses.replace(cp, needs_layout_passes=False)
    return cp


def _k1_body(x_ref, m_ref, acc_ref):
    c = pl.program_id(1)
    blk = x_ref[0]
    m = blk[0:_G]
    for k in range(1, _CH // _G):
        m = jnp.maximum(m, blk[k * _G:(k + 1) * _G])

    @pl.when(c == 0)
    def _():
        acc_ref[...] = m

    @pl.when(c > 0)
    def _():
        acc_ref[...] = jnp.maximum(acc_ref[...], m)

    @pl.when(c == pl.num_programs(1) - 1)
    def _():
        m_ref[0] = jnp.maximum(acc_ref[...], m)


def _k1(x):
    return pl.pallas_call(
        _k1_body,
        grid=(_B, _S // _CH),
        in_specs=[pl.BlockSpec((1, _CH, _F), lambda i, c: (i, c, 0))],
        out_specs=pl.BlockSpec((1, _G, _F), lambda i, c: (i, 0, 0)),
        out_shape=jax.ShapeDtypeStruct((_B, _G, _F), x.dtype),
        scratch_shapes=[pltpu.VMEM((_G, _F), x.dtype)],
        compiler_params=pltpu.CompilerParams(
            dimension_semantics=("arbitrary", "arbitrary")),
    )(x)


def _k2_body(m_ref, q_ref):
    w = jnp.concatenate([m_ref[0], m_ref[1]], axis=1)
    rows = lax.broadcasted_iota(jnp.int32, w.shape, 0)
    idx_rows = []
    for _ in range(_K):
        m = jnp.max(w, axis=0, keepdims=True)
        idx = jnp.where(w == m, rows, _G)
        amin = jnp.min(idx, axis=0, keepdims=True)
        idx_rows.append(amin)
        w = jnp.where(rows == amin, _NEG, w)
    q_ref[0] = jnp.concatenate(idx_rows, axis=0)


def _k2(m):
    return pl.pallas_call(
        _k2_body,
        grid=(_B // 2,),
        in_specs=[pl.BlockSpec((2, _G, _F), lambda i: (i, 0, 0))],
        out_specs=pl.BlockSpec((1, _K, 2 * _F), lambda i: (i, 0, 0)),
        out_shape=jax.ShapeDtypeStruct((_B // 2, _K, 2 * _F), jnp.int32),
        compiler_params=pltpu.CompilerParams(
            dimension_semantics=("arbitrary",)),
    )(m)


def _sc_topk(xg, qp):
    info = plsc.get_sparse_core_info()
    nw = info.num_cores * info.num_subcores
    ppw = _NPAIR // nw
    mesh = plsc.VectorSubcoreMesh(core_axis_name="c", subcore_axis_name="s")

    @functools.partial(
        pl.kernel, mesh=mesh, compiler_params=_sc_params(),
        out_type=jax.ShapeDtypeStruct((_NPAIR * _GSZ,), jnp.float32),
        scratch_types=[
            pltpu.VMEM((ppw * _GSZ,), jnp.int32),
            pltpu.VMEM((_CAND,), jnp.int32),
            pltpu.VMEM((_CAND, _GSZ), jnp.float32),
            pltpu.VMEM((ppw * _GSZ,), jnp.float32),
            pltpu.SemaphoreType.DMA,
        ],
    )
    def k(xg_hbm, q_hbm, out_hbm, q_v, gid_v, rows_v, res_v, sem):
        wid = lax.axis_index("s") * info.num_cores + lax.axis_index("c")
        base = wid * ppw * _GSZ
        pltpu.sync_copy(q_hbm.at[pl.ds(base, ppw * _GSZ)], q_v)
        iota = lax.iota(jnp.int32, _GSZ)
        zero = jnp.zeros((_GSZ,), jnp.int32)

        @pl.loop(0, ppw)
        def _(local):
            pair = wid * ppw + local
            b = pair >> 6
            f = pair & (_F - 1)
            gbase = b * (_S * _F // _GSZ) + (f >> 4)
            qvec = q_v[pl.ds(local * _GSZ, _GSZ)]
            for r in range(_K):
                gid = qvec[r]
                gid_v[pl.ds(r * _GSZ, _GSZ)] = (
                    gbase + gid * (_F // _GSZ) + (_G * _F // _GSZ) * iota)
            pltpu.async_copy(xg_hbm.at[gid_v], rows_v, sem).wait()
            lane = jnp.full((_GSZ,), f & (_GSZ - 1), jnp.int32)
            acc = None
            for t in range(_K):
                cand = plsc.load_gather(rows_v, [t * _GSZ + iota, lane])
                sc, _ = plsc.sort_key_val(cand, zero, descending=True)
                if acc is None:
                    acc = sc
                else:
                    hi = jnp.maximum(acc, lax.rev(sc, (0,)))
                    acc, _ = plsc.sort_key_val(hi, zero, descending=True)
            res_v[pl.ds(local * _GSZ, _GSZ)] = acc

        pltpu.sync_copy(res_v, out_hbm.at[pl.ds(base, ppw * _GSZ)])

    return k(xg, qp)


def kernel(x):
    b, s, f = x.shape
    assert (b, s, f) == (_B, _S, _F)
    m = _k1(x)
    q = _k2(m)
    qp = q.reshape(_B // 2, _K, 2, _F).transpose(0, 2, 3, 1).reshape(
        _NPAIR, _K)
    qp = jnp.concatenate([qp, qp], axis=1).reshape(-1)
    xg = x.reshape(_B * _S * _F // _GSZ, _GSZ)
    out = _sc_topk(xg, qp)
    return out.reshape(_B, _F, _GSZ)[:, :, :_K].transpose(0, 2, 1)


# speedup vs baseline: 90.3972x; 2.5338x over previous
import dataclasses
import functools

import jax
import jax.numpy as jnp
from jax import lax
from jax.experimental import pallas as pl
from jax.experimental.pallas import tpu as pltpu
from jax.experimental.pallas import tpu_sc as plsc

_K = 8
_NEG = float("-inf")

_B, _S, _F = 32, 32768, 64
_G = 2048
_GSZ = _S // _G
_CH = 8192
_NPAIR = _B * _F
_CAND = _K * _GSZ


def _sc_params():
    cp = pltpu.CompilerParams(use_tc_tiling_on_sc=False)
    if "needs_layout_passes" in pltpu.CompilerParams.__dataclass_fields__:
        cp = dataclasses.replace(cp, needs_layout_passes=False)
    return cp


def _k1_body(x_ref, m_ref, acc_ref):
    c = pl.program_id(1)
    blk = x_ref[0]
    m = blk[:, 0:_G]
    for k in range(1, _CH // _G):
        m = jnp.maximum(m, blk[:, k * _G:(k + 1) * _G])

    @pl.when(c == 0)
    def _():
        acc_ref[...] = m

    @pl.when(c > 0)
    def _():
        acc_ref[...] = jnp.maximum(acc_ref[...], m)

    @pl.when(c == pl.num_programs(1) - 1)
    def _():
        m_ref[0] = jnp.maximum(acc_ref[...], m)


def _k1(x):
    return pl.pallas_call(
        _k1_body,
        grid=(_B, _S // _CH),
        in_specs=[pl.BlockSpec((1, _F, _CH), lambda i, c: (i, 0, c))],
        out_specs=pl.BlockSpec((1, _F, _G), lambda i, c: (i, 0, 0)),
        out_shape=jax.ShapeDtypeStruct((_B, _F, _G), x.dtype),
        scratch_shapes=[pltpu.VMEM((_F, _G), x.dtype)],
        compiler_params=pltpu.CompilerParams(
            dimension_semantics=("arbitrary", "arbitrary")),
    )(x)


def _k2_body(m_ref, q_ref):
    w = jnp.concatenate([m_ref[0], m_ref[1]], axis=1)
    rows = lax.broadcasted_iota(jnp.int32, w.shape, 0)
    idx_rows = []
    for _ in range(_K):
        m = jnp.max(w, axis=0, keepdims=True)
        idx = jnp.where(w == m, rows, _G)
        amin = jnp.min(idx, axis=0, keepdims=True)
        idx_rows.append(amin)
        w = jnp.where(rows == amin, _NEG, w)
    q_ref[0] = jnp.concatenate(idx_rows, axis=0)


def _k2(m):
    return pl.pallas_call(
        _k2_body,
        grid=(_B // 2,),
        in_specs=[pl.BlockSpec((2, _G, _F), lambda i: (i, 0, 0))],
        out_specs=pl.BlockSpec((1, _K, 2 * _F), lambda i: (i, 0, 0)),
        out_shape=jax.ShapeDtypeStruct((_B // 2, _K, 2 * _F), jnp.int32),
        compiler_params=pltpu.CompilerParams(
            dimension_semantics=("arbitrary",)),
    )(m)


def _sc_topk(xg, qp):
    info = plsc.get_sparse_core_info()
    nw = info.num_cores * info.num_subcores
    ppw = _NPAIR // nw
    mesh = plsc.VectorSubcoreMesh(core_axis_name="c", subcore_axis_name="s")

    @functools.partial(
        pl.kernel, mesh=mesh, compiler_params=_sc_params(),
        out_type=jax.ShapeDtypeStruct((_NPAIR * _GSZ,), jnp.float32),
        scratch_types=[
            pltpu.VMEM((ppw * _GSZ,), jnp.int32),
            pltpu.VMEM((_CAND,), jnp.int32),
            pltpu.VMEM((_CAND, _GSZ), jnp.float32),
            pltpu.VMEM((ppw * _GSZ,), jnp.float32),
            pltpu.SemaphoreType.DMA,
        ],
    )
    def k(xg_hbm, q_hbm, out_hbm, q_v, gid_v, rows_v, res_v, sem):
        wid = lax.axis_index("s") * info.num_cores + lax.axis_index("c")
        base = wid * ppw * _GSZ
        pltpu.sync_copy(q_hbm.at[pl.ds(base, ppw * _GSZ)], q_v)
        iota = lax.iota(jnp.int32, _GSZ)
        zero = jnp.zeros((_GSZ,), jnp.int32)

        @pl.loop(0, ppw)
        def _(local):
            pair = wid * ppw + local
            gbase = pair * (_S // _GSZ)
            qvec = q_v[pl.ds(local * _GSZ, _GSZ)]
            for r in range(_K):
                gid = qvec[r]
                gid_v[pl.ds(r * _GSZ, _GSZ)] = (
                    gbase + (gid >> 4) + (_G // _GSZ) * iota)
            pltpu.async_copy(xg_hbm.at[gid_v], rows_v, sem).wait()
            acc = None
            for t in range(_K):
                lane = jnp.full((_GSZ,), qvec[t] & (_GSZ - 1), jnp.int32)
                cand = plsc.load_gather(rows_v, [t * _GSZ + iota, lane])
                sc, _ = plsc.sort_key_val(cand, zero, descending=True)
                if acc is None:
                    acc = sc
                else:
                    hi = jnp.maximum(acc, lax.rev(sc, (0,)))
                    acc, _ = plsc.sort_key_val(hi, zero, descending=True)
            res_v[pl.ds(local * _GSZ, _GSZ)] = acc

        pltpu.sync_copy(res_v, out_hbm.at[pl.ds(base, ppw * _GSZ)])

    return k(xg, qp)


def kernel(x):
    b, s, f = x.shape
    assert (b, s, f) == (_B, _S, _F)
    xt = jnp.transpose(x, (0, 2, 1))
    m = _k1(xt)
    q = _k2(jnp.transpose(m, (0, 2, 1)))
    qp = q.reshape(_B // 2, _K, 2, _F).transpose(0, 2, 3, 1).reshape(
        _NPAIR, _K)
    qp = jnp.concatenate([qp, qp], axis=1).reshape(-1)
    xg = xt.reshape(_B * _S * _F // _GSZ, _GSZ)
    out = _sc_topk(xg, qp)
    return out.reshape(_B, _F, _GSZ)[:, :, :_K].transpose(0, 2, 1)


# speedup vs baseline: 105.2342x; 1.1641x over previous
import dataclasses
import functools

import jax
import jax.numpy as jnp
from jax import lax
from jax.experimental import pallas as pl
from jax.experimental.pallas import tpu as pltpu
from jax.experimental.pallas import tpu_sc as plsc

_K = 8
_NEG = float("-inf")

_B, _S, _F = 32, 32768, 64
_G = 2048
_GSZ = _S // _G
_CH = 8192
_NPAIR = _B * _F
_CAND = _K * _GSZ


def _sc_params():
    cp = pltpu.CompilerParams(use_tc_tiling_on_sc=False)
    if "needs_layout_passes" in pltpu.CompilerParams.__dataclass_fields__:
        cp = dataclasses.replace(cp, needs_layout_passes=False)
    return cp


def _k1_body(x_ref, y_ref, m_ref, acc_ref):
    c = pl.program_id(1)
    blk = x_ref[0]
    y_ref[...] = blk.reshape(_F, _CH // 128, 128)
    m = blk[:, 0:_G]
    for k in range(1, _CH // _G):
        m = jnp.maximum(m, blk[:, k * _G:(k + 1) * _G])

    @pl.when(c == 0)
    def _():
        acc_ref[...] = m

    @pl.when(c > 0)
    def _():
        acc_ref[...] = jnp.maximum(acc_ref[...], m)

    @pl.when(c == pl.num_programs(1) - 1)
    def _():
        m_ref[0] = jnp.maximum(acc_ref[...], m)


def _k1(x):
    return pl.pallas_call(
        _k1_body,
        grid=(_B, _S // _CH),
        in_specs=[pl.BlockSpec((1, _F, _CH), lambda i, c: (i, 0, c))],
        out_specs=[
            pl.BlockSpec((_F, _CH // 128, 128), lambda i, c: (i, c, 0)),
            pl.BlockSpec((1, _F, _G), lambda i, c: (i, 0, 0)),
        ],
        out_shape=[
            jax.ShapeDtypeStruct((_B * _F, _S // 128, 128), x.dtype),
            jax.ShapeDtypeStruct((_B, _F, _G), x.dtype),
        ],
        scratch_shapes=[pltpu.VMEM((_F, _G), x.dtype)],
        compiler_params=pltpu.CompilerParams(
            dimension_semantics=("arbitrary", "arbitrary")),
    )(x)


def _k2_body(m_ref, q_ref):
    w = jnp.concatenate([m_ref[0], m_ref[1]], axis=1)
    rows = lax.broadcasted_iota(jnp.int32, w.shape, 0)
    idx_rows = []
    for _ in range(_K):
        m = jnp.max(w, axis=0, keepdims=True)
        idx = jnp.where(w == m, rows, _G)
        amin = jnp.min(idx, axis=0, keepdims=True)
        idx_rows.append(amin)
        w = jnp.where(rows == amin, _NEG, w)
    q_ref[0] = jnp.concatenate(idx_rows, axis=0)


def _k2(m):
    return pl.pallas_call(
        _k2_body,
        grid=(_B // 2,),
        in_specs=[pl.BlockSpec((2, _G, _F), lambda i: (i, 0, 0))],
        out_specs=pl.BlockSpec((1, _K, 2 * _F), lambda i: (i, 0, 0)),
        out_shape=jax.ShapeDtypeStruct((_B // 2, _K, 2 * _F), jnp.int32),
        compiler_params=pltpu.CompilerParams(
            dimension_semantics=("arbitrary",)),
    )(m)


def _sc_topk(xg, qp):
    info = plsc.get_sparse_core_info()
    nw = info.num_cores * info.num_subcores
    ppw = _NPAIR // nw
    mesh = plsc.VectorSubcoreMesh(core_axis_name="c", subcore_axis_name="s")

    @functools.partial(
        pl.kernel, mesh=mesh, compiler_params=_sc_params(),
        out_type=jax.ShapeDtypeStruct((_NPAIR * _GSZ,), jnp.float32),
        scratch_types=[
            pltpu.VMEM((ppw * _GSZ,), jnp.int32),
            pltpu.VMEM((_CAND,), jnp.int32),
            pltpu.VMEM((_CAND, _GSZ), jnp.float32),
            pltpu.VMEM((ppw * _GSZ,), jnp.float32),
            pltpu.SemaphoreType.DMA,
        ],
    )
    def k(xg_hbm, q_hbm, out_hbm, q_v, gid_v, rows_v, res_v, sem):
        wid = lax.axis_index("s") * info.num_cores + lax.axis_index("c")
        base = wid * ppw * _GSZ
        pltpu.sync_copy(q_hbm.at[pl.ds(base, ppw * _GSZ)], q_v)
        iota = lax.iota(jnp.int32, _GSZ)
        zero = jnp.zeros((_GSZ,), jnp.int32)

        @pl.loop(0, ppw)
        def _(local):
            pair = wid * ppw + local
            gbase = pair * (_S // _GSZ)
            qvec = q_v[pl.ds(local * _GSZ, _GSZ)]
            for r in range(_K):
                gid = qvec[r]
                gid_v[pl.ds(r * _GSZ, _GSZ)] = (
                    gbase + (gid >> 4) + (_G // _GSZ) * iota)
            pltpu.async_copy(xg_hbm.at[gid_v], rows_v, sem).wait()
            acc = None
            for t in range(_K):
                lane = jnp.full((_GSZ,), qvec[t] & (_GSZ - 1), jnp.int32)
                cand = plsc.load_gather(rows_v, [t * _GSZ + iota, lane])
                sc, _ = plsc.sort_key_val(cand, zero, descending=True)
                if acc is None:
                    acc = sc
                else:
                    hi = jnp.maximum(acc, lax.rev(sc, (0,)))
                    acc, _ = plsc.sort_key_val(hi, zero, descending=True)
            res_v[pl.ds(local * _GSZ, _GSZ)] = acc

        pltpu.sync_copy(res_v, out_hbm.at[pl.ds(base, ppw * _GSZ)])

    return k(xg, qp)


def kernel(x):
    b, s, f = x.shape
    assert (b, s, f) == (_B, _S, _F)
    xt = jnp.transpose(x, (0, 2, 1))
    ys, m = _k1(xt)
    q = _k2(jnp.transpose(m, (0, 2, 1)))
    qp = q.reshape(_B // 2, _K, 2, _F).transpose(0, 2, 3, 1).reshape(
        _NPAIR, _K)
    qp = jnp.concatenate([qp, qp], axis=1).reshape(-1)
    xg = ys.reshape(_B * _S * _F // _GSZ, _GSZ)
    out = _sc_topk(xg, qp)
    return out.reshape(_B, _F, _GSZ)[:, :, :_K].transpose(0, 2, 1)
